# baseline (speedup 1.0000x reference)
import functools
import jax
import jax.numpy as jnp
from jax import lax
from jax.experimental import pallas as pl
from jax.experimental.pallas import tpu as pltpu
from jax.experimental.pallas import tpu_sc as plsc

EMB_DIM = 64
NUM_CORES = 2
NUM_SUBCORES = 16
NUM_WORKERS = NUM_CORES * NUM_SUBCORES
CHUNK = 128
LANES = 16


def _make_sc_kernel(B, D):
    assert B % (NUM_WORKERS * CHUNK) == 0
    b_per_w = B // NUM_WORKERS
    n_chunks = b_per_w // CHUNK

    mesh = plsc.VectorSubcoreMesh(core_axis_name="c", subcore_axis_name="s")

    @functools.partial(
        pl.kernel,
        out_type=jax.ShapeDtypeStruct((B, D), jnp.float32),
        mesh=mesh,
        scratch_types=[
            pltpu.VMEM((n_chunks, CHUNK), jnp.int32),
            pltpu.VMEM((CHUNK, D), jnp.float32),
            pltpu.SemaphoreType.DMA,
        ],
        compiler_params=pltpu.CompilerParams(use_tc_tiling_on_sc=False),
    )
    def emb_kernel(tokens_hbm, table_hbm, out_hbm, idx_v, rows_v, sem):
        wid = lax.axis_index("s") * NUM_CORES + lax.axis_index("c")
        pltpu.sync_copy(tokens_hbm.at[pl.ds(wid * n_chunks, n_chunks)], idx_v)

        def chunk_body(k, carry):
            pltpu.async_copy(table_hbm.at[idx_v.at[k]], rows_v, sem).wait()

            def scale_row(i, c):
                for j in range(D // LANES):
                    sl = pl.ds(j * LANES, LANES)
                    rows_v[i, sl] = rows_v[i, sl] * 8.0
                return c

            lax.fori_loop(0, CHUNK, scale_row, 0, unroll=4)

            base = wid * b_per_w + k * CHUNK
            pltpu.sync_copy(rows_v, out_hbm.at[pl.ds(base, CHUNK)])
            return carry

        lax.fori_loop(0, n_chunks, chunk_body, 0)

    return emb_kernel


@jax.jit
def kernel(tokens, emb_table):
    B = tokens.shape[0] * tokens.shape[1]
    D = emb_table.shape[1]
    flat = tokens.reshape(NUM_WORKERS * (B // (NUM_WORKERS * CHUNK)), CHUNK)
    flat = flat.astype(jnp.int32)
    out = _make_sc_kernel(B, D)(flat, emb_table)
    return out.reshape(tokens.shape + (D,))


# speedup vs baseline: 1.1401x; 1.1401x over previous
import functools
import jax
import jax.numpy as jnp
from jax import lax
from jax.experimental import pallas as pl
from jax.experimental.pallas import tpu as pltpu
from jax.experimental.pallas import tpu_sc as plsc

EMB_DIM = 64
NUM_CORES = 2
NUM_SUBCORES = 16
NUM_WORKERS = NUM_CORES * NUM_SUBCORES
CHUNK = 128
LANES = 16
NBUF = 4
AHEAD = 2


def _make_sc_kernel(B, D):
    assert B % (NUM_WORKERS * CHUNK * NBUF) == 0
    b_per_w = B // NUM_WORKERS
    n_chunks = b_per_w // CHUNK

    mesh = plsc.VectorSubcoreMesh(core_axis_name="c", subcore_axis_name="s")

    @functools.partial(
        pl.kernel,
        out_type=jax.ShapeDtypeStruct((B, D), jnp.float32),
        mesh=mesh,
        scratch_types=[
            pltpu.VMEM((n_chunks, CHUNK), jnp.int32),
            [pltpu.VMEM((CHUNK, D), jnp.float32) for _ in range(NBUF)],
            [pltpu.SemaphoreType.DMA for _ in range(NBUF)],
            [pltpu.SemaphoreType.DMA for _ in range(NBUF)],
        ],
        compiler_params=pltpu.CompilerParams(use_tc_tiling_on_sc=False),
    )
    def emb_kernel(tokens_hbm, table_hbm, out_hbm, idx_v, rows, gsems, wsems):
        wid = lax.axis_index("s") * NUM_CORES + lax.axis_index("c")
        base = wid * b_per_w
        pltpu.sync_copy(tokens_hbm.at[pl.ds(wid * n_chunks, n_chunks)], idx_v)

        for k in range(AHEAD):
            pltpu.async_copy(table_hbm.at[idx_v.at[k]], rows[k], gsems[k])

        def scale(buf):
            def scale_row(i, c):
                for j in range(D // LANES):
                    sl = pl.ds(j * LANES, LANES)
                    buf[i, sl] = buf[i, sl] * 8.0
                return c

            lax.fori_loop(0, CHUNK, scale_row, 0, unroll=4)

        def body(g, carry):
            for b in range(NBUF):
                k = g * NBUF + b
                pltpu.make_async_copy(
                    table_hbm.at[idx_v.at[k]], rows[b], gsems[b]
                ).wait()
                scale(rows[b])
                pltpu.async_copy(
                    rows[b], out_hbm.at[pl.ds(base + k * CHUNK, CHUNK)],
                    wsems[b],
                )
                nb = (b + AHEAD) % NBUF
                kn = k + AHEAD

                @pl.when(kn < n_chunks)
                def _():
                    @pl.when(kn >= NBUF)
                    def _():
                        pltpu.make_async_copy(
                            rows[nb],
                            out_hbm.at[pl.ds(base + (kn - NBUF) * CHUNK, CHUNK)],
                            wsems[nb],
                        ).wait()

                    pltpu.async_copy(
                        table_hbm.at[idx_v.at[kn]], rows[nb], gsems[nb]
                    )

            return carry

        lax.fori_loop(0, n_chunks // NBUF, body, 0)

        for b in range(NBUF):
            k = n_chunks - NBUF + b
            pltpu.make_async_copy(
                rows[b], out_hbm.at[pl.ds(base + k * CHUNK, CHUNK)],
                wsems[b],
            ).wait()

    return emb_kernel


@jax.jit
def kernel(tokens, emb_table):
    B = tokens.shape[0] * tokens.shape[1]
    D = emb_table.shape[1]
    flat = tokens.reshape(NUM_WORKERS * (B // (NUM_WORKERS * CHUNK)), CHUNK)
    flat = flat.astype(jnp.int32)
    out = _make_sc_kernel(B, D)(flat, emb_table)
    return out.reshape(tokens.shape + (D,))
